# baseline (speedup 1.0000x reference)
import functools

import jax
import jax.numpy as jnp
from jax import lax
from jax.experimental import pallas as pl
from jax.experimental.pallas import tpu as pltpu
from jax.experimental.pallas import tpu_sc as plsc

_NC = 2
_NS = 16
_NW = _NC * _NS


def _pick_chunk(ept):
    for k in range(128, 7, -8):
        if ept % k == 0:
            return k
    raise ValueError(f"no valid chunk size for {ept} edges per tile")


def _make_deg(n_pad, ept):
    mesh = plsc.VectorSubcoreMesh(core_axis_name="c", subcore_axis_name="s")
    ngrp = ept // 16

    @functools.partial(
        pl.kernel,
        mesh=mesh,
        out_type=jax.ShapeDtypeStruct((_NW, 1, n_pad), jnp.float32),
        scratch_types=[
            pltpu.VMEM((ept,), jnp.int32),
            pltpu.VMEM((n_pad,), jnp.float32),
        ],
        compiler_params=pltpu.CompilerParams(needs_layout_passes=False),
    )
    def deg(dst_hbm, zero_hbm, out_hbm, dst_v, hist):
        cid = lax.axis_index("c")
        sid = lax.axis_index("s")
        wid = sid * _NC + cid
        pltpu.sync_copy(zero_hbm, hist)
        pltpu.sync_copy(dst_hbm.at[pl.ds(wid * ept, ept)], dst_v)
        ones16 = jnp.ones((16,), jnp.float32)

        def body(g, c):
            idx = dst_v[pl.ds(g * 16, 16)]
            plsc.addupdate_scatter(hist, [idx], ones16)
            return c

        lax.fori_loop(0, ngrp, body, 0)
        pltpu.sync_copy(hist, out_hbm.at[wid, 0])

    return deg


def _make_agg(n_pad, d, nchunk, k):
    rpt = n_pad // _NS
    mesh = plsc.VectorSubcoreMesh(core_axis_name="c", subcore_axis_name="s")

    ept = nchunk * k

    @functools.partial(
        pl.kernel,
        mesh=mesh,
        out_type=jax.ShapeDtypeStruct((_NC, n_pad, d), jnp.float32),
        scratch_types=[
            pltpu.VMEM((nchunk, k), jnp.int32),
            pltpu.VMEM((k,), jnp.int32),
            pltpu.VMEM((k,), jnp.int32),
            pltpu.VMEM((k, d), jnp.float32),
            pltpu.VMEM((k, d), jnp.float32),
            pltpu.VMEM_SHARED((n_pad, d), jnp.float32),
            pltpu.SemaphoreType.DMA,
            pltpu.SemaphoreType.DMA,
        ],
    )
    def agg(src_hbm, dst_hbm, h_hbm, zero_hbm, out_hbm,
            dst_v, stage_a, stage_b, rows_a, rows_b, acc, sem_a, sem_b):
        cid = lax.axis_index("c")
        sid = lax.axis_index("s")
        wid = sid * _NC + cid
        r0 = sid * rpt
        base = wid * ept
        pltpu.sync_copy(zero_hbm.at[pl.ds(r0, rpt)], acc.at[pl.ds(r0, rpt)])
        pltpu.sync_copy(dst_hbm.at[wid], dst_v)
        plsc.subcore_barrier()

        def body(j, c):
            pltpu.sync_copy(src_hbm.at[pl.ds(base + j * k, k)], stage_a)
            pltpu.async_copy(h_hbm.at[stage_a], rows_a, sem_a).wait()
            pltpu.sync_copy(rows_a, acc.at[dst_v.at[j]], add=True)
            return c

        lax.fori_loop(0, nchunk, body, 0)

        plsc.subcore_barrier()
        pltpu.sync_copy(acc.at[pl.ds(r0, rpt)], out_hbm.at[cid, pl.ds(r0, rpt)])

    return agg


def _dinv_of(deg_ref):
    dsum = jnp.sum(deg_ref[...], axis=0, keepdims=True) + 1.0
    dinv_row = lax.rsqrt(dsum)
    one = jnp.ones((1, 1), jnp.float32)
    return lax.dot_general(dinv_row, one, (((0,), (0,)), ((), ())),
                           preferred_element_type=jnp.float32)


def _mm1_body(x_ref, w_ref, deg_ref, out_ref):
    dinv = _dinv_of(deg_ref)
    out_ref[...] = jnp.dot(x_ref[...], w_ref[...],
                           preferred_element_type=jnp.float32) * dinv


def _mm2_body(a0_ref, a1_ref, hp_ref, deg_ref, b_ref, w_ref, out_ref):
    dinv = _dinv_of(deg_ref)
    h1 = (a0_ref[...] + a1_ref[...] + hp_ref[...]) * dinv + b_ref[...]
    h1 = jnp.maximum(h1, 0.0)
    out_ref[...] = jnp.dot(h1, w_ref[...],
                           preferred_element_type=jnp.float32) * dinv


def _make_z_body(dout):
    def _z_body(a0_ref, a1_ref, hp_ref, deg_ref, b_ref,
                zfull_ref, z_ref):
        dinv = _dinv_of(deg_ref)
        zf = (a0_ref[...] + a1_ref[...] + hp_ref[...]) * dinv + b_ref[...]
        zfull_ref[...] = zf
        z_ref[...] = zf[:, :dout]
    return _z_body


def _dec_body(za_ref, zb_ref, out_ref):
    s = lax.dot_general(za_ref[...], zb_ref[...],
                        (((1,), (1,)), ((), ())),
                        preferred_element_type=jnp.float32)
    out_ref[...] = jax.nn.sigmoid(s)


def kernel(x, edge_index, W1, b1, W2, b2):
    n, din = x.shape
    dhid = W1.shape[1]
    dout = W2.shape[1]
    e = edge_index.shape[1]
    ept = e // _NW
    k = _pick_chunk(ept)
    nchunk = ept // k

    n_pad = ((n + 127) // 128) * 128

    src = edge_index[0]
    dst_flat = edge_index[1]
    dst = edge_index[1].reshape(_NW, nchunk, k)
    z1r = jnp.zeros((n_pad,), jnp.float32)
    zhid = jnp.zeros((n_pad, dhid), jnp.float32)
    b1r = b1.reshape(1, dhid)
    b2r = b2.reshape(1, dout)
    d2 = max(dout, 128)
    W2p = jnp.pad(W2, ((0, 0), (0, d2 - dout))) if d2 != dout else W2
    b2p = jnp.pad(b2r, ((0, 0), (0, d2 - dout))) if d2 != dout else b2r

    deg32 = _make_deg(n_pad, ept)(dst_flat, z1r).reshape(_NW, n_pad)

    R = 1024
    grid = (pl.cdiv(n, R),)
    row = lambda i: (i, 0)
    fixed = lambda i: (0, 0)
    dspec = pl.BlockSpec((_NW, R), lambda i: (0, i))

    h1p = pl.pallas_call(
        _mm1_body,
        grid=grid,
        in_specs=[
            pl.BlockSpec((R, din), row),
            pl.BlockSpec((din, dhid), fixed),
            dspec,
        ],
        out_specs=pl.BlockSpec((R, dhid), row),
        out_shape=jax.ShapeDtypeStruct((n, dhid), jnp.float32),
    )(x, W1, deg32)

    agg1 = _make_agg(n_pad, dhid, nchunk, k)(src, dst, h1p, zhid)
    a10, a11 = agg1[0], agg1[1]

    h2p = pl.pallas_call(
        _mm2_body,
        grid=grid,
        in_specs=[
            pl.BlockSpec((R, dhid), row),
            pl.BlockSpec((R, dhid), row),
            pl.BlockSpec((R, dhid), row),
            dspec,
            pl.BlockSpec((1, dhid), fixed),
            pl.BlockSpec((dhid, d2), fixed),
        ],
        out_specs=pl.BlockSpec((R, d2), row),
        out_shape=jax.ShapeDtypeStruct((n, d2), jnp.float32),
    )(a10, a11, h1p, deg32, b1r, W2p)

    zo2 = jnp.zeros((n_pad, d2), jnp.float32)
    agg2 = _make_agg(n_pad, d2, nchunk, k)(src, dst, h2p, zo2)
    a20, a21 = agg2[0], agg2[1]

    zfull, z = pl.pallas_call(
        _make_z_body(dout),
        grid=grid,
        in_specs=[
            pl.BlockSpec((R, d2), row),
            pl.BlockSpec((R, d2), row),
            pl.BlockSpec((R, d2), row),
            dspec,
            pl.BlockSpec((1, d2), fixed),
        ],
        out_specs=[pl.BlockSpec((R, d2), row), pl.BlockSpec((R, dout), row)],
        out_shape=[jax.ShapeDtypeStruct((n, d2), jnp.float32),
                   jax.ShapeDtypeStruct((n, dout), jnp.float32)],
    )(a20, a21, h2p, deg32, b2p)

    BR = 512
    gdec = pl.cdiv(n, BR)
    adj = pl.pallas_call(
        _dec_body,
        grid=(gdec, gdec),
        in_specs=[
            pl.BlockSpec((BR, d2), lambda i, j: (i, 0)),
            pl.BlockSpec((BR, d2), lambda i, j: (j, 0)),
        ],
        out_specs=pl.BlockSpec((BR, BR), lambda i, j: (i, j)),
        out_shape=jax.ShapeDtypeStruct((n, n), jnp.float32),
    )(zfull, zfull)

    return (adj, z)


# speedup vs baseline: 1.2572x; 1.2572x over previous
import functools

import jax
import jax.numpy as jnp
from jax import lax
from jax.experimental import pallas as pl
from jax.experimental.pallas import tpu as pltpu
from jax.experimental.pallas import tpu_sc as plsc

_NC = 2
_NS = 16
_NW = _NC * _NS


def _pick_chunk(ept):
    for k in range(128, 7, -8):
        if ept % k == 0:
            return k
    raise ValueError(f"no valid chunk size for {ept} edges per tile")


def _make_deg(n_pad, ept):
    mesh = plsc.VectorSubcoreMesh(core_axis_name="c", subcore_axis_name="s")
    ngrp = ept // 16

    @functools.partial(
        pl.kernel,
        mesh=mesh,
        out_type=jax.ShapeDtypeStruct((_NW, 1, n_pad), jnp.float32),
        scratch_types=[
            pltpu.VMEM((ept,), jnp.int32),
            pltpu.VMEM((n_pad,), jnp.float32),
        ],
        compiler_params=pltpu.CompilerParams(needs_layout_passes=False),
    )
    def deg(dst_hbm, zero_hbm, out_hbm, dst_v, hist):
        cid = lax.axis_index("c")
        sid = lax.axis_index("s")
        wid = sid * _NC + cid
        pltpu.sync_copy(zero_hbm, hist)
        pltpu.sync_copy(dst_hbm.at[pl.ds(wid * ept, ept)], dst_v)
        ones16 = jnp.ones((16,), jnp.float32)

        def body(g, c):
            idx = dst_v[pl.ds(g * 16, 16)]
            plsc.addupdate_scatter(hist, [idx], ones16)
            return c

        lax.fori_loop(0, ngrp, body, 0)
        pltpu.sync_copy(hist, out_hbm.at[wid, 0])

    return deg


def _make_agg(n_pad, d, nchunk, k):
    rpt = n_pad // _NS
    mesh = plsc.VectorSubcoreMesh(core_axis_name="c", subcore_axis_name="s")

    ept = nchunk * k

    @functools.partial(
        pl.kernel,
        mesh=mesh,
        out_type=jax.ShapeDtypeStruct((_NC, n_pad, d), jnp.float32),
        scratch_types=[
            pltpu.VMEM((nchunk, k), jnp.int32),
            pltpu.VMEM((k,), jnp.int32),
            pltpu.VMEM((k,), jnp.int32),
            pltpu.VMEM((k, d), jnp.float32),
            pltpu.VMEM((k, d), jnp.float32),
            pltpu.VMEM_SHARED((n_pad, d), jnp.float32),
            pltpu.SemaphoreType.DMA,
            pltpu.SemaphoreType.DMA,
            pltpu.SemaphoreType.DMA,
            pltpu.SemaphoreType.DMA,
        ],
    )
    def agg(src_hbm, dst_hbm, h_hbm, zero_hbm, out_hbm,
            dst_v, stage_a, stage_b, rows_a, rows_b, acc,
            sem_a, sem_b, sem_ia, sem_ib):
        cid = lax.axis_index("c")
        sid = lax.axis_index("s")
        wid = sid * _NC + cid
        r0 = sid * rpt
        base = wid * ept
        pltpu.sync_copy(zero_hbm.at[pl.ds(r0, rpt)], acc.at[pl.ds(r0, rpt)])
        pltpu.sync_copy(dst_hbm.at[wid], dst_v)
        plsc.subcore_barrier()

        def sidx(j):
            return src_hbm.at[pl.ds(base + j * k, k)]

        if nchunk % 2 == 1 and nchunk >= 3:
            pltpu.sync_copy(sidx(0), stage_a)
            pltpu.async_copy(h_hbm.at[stage_a], rows_a, sem_a)
            pltpu.async_copy(sidx(1), stage_b, sem_ib)
            last = nchunk - 1

            def body(i, c):
                j0 = 2 * i
                pltpu.make_async_copy(sidx(j0 + 1), stage_b, sem_ib).wait()
                pltpu.make_async_copy(h_hbm.at[stage_a], rows_a, sem_a).wait()
                pltpu.async_copy(h_hbm.at[stage_b], rows_b, sem_b)
                pltpu.async_copy(sidx(j0 + 2), stage_a, sem_ia)
                pltpu.sync_copy(rows_a, acc.at[dst_v.at[j0]], add=True)
                pltpu.make_async_copy(sidx(j0 + 2), stage_a, sem_ia).wait()
                pltpu.make_async_copy(h_hbm.at[stage_b], rows_b, sem_b).wait()
                pltpu.async_copy(h_hbm.at[stage_a], rows_a, sem_a)
                j3 = jnp.minimum(j0 + 3, last)
                pltpu.async_copy(sidx(j3), stage_b, sem_ib)
                pltpu.sync_copy(rows_b, acc.at[dst_v.at[j0 + 1]], add=True)
                return c

            lax.fori_loop(0, (nchunk - 1) // 2, body, 0)
            pltpu.make_async_copy(h_hbm.at[stage_a], rows_a, sem_a).wait()
            pltpu.make_async_copy(sidx(last), stage_b, sem_ib).wait()
            pltpu.sync_copy(rows_a, acc.at[dst_v.at[last]], add=True)
        else:
            def body(j, c):
                pltpu.sync_copy(sidx(j), stage_a)
                pltpu.async_copy(h_hbm.at[stage_a], rows_a, sem_a).wait()
                pltpu.sync_copy(rows_a, acc.at[dst_v.at[j]], add=True)
                return c

            lax.fori_loop(0, nchunk, body, 0)

        plsc.subcore_barrier()
        pltpu.sync_copy(acc.at[pl.ds(r0, rpt)], out_hbm.at[cid, pl.ds(r0, rpt)])

    return agg


def _dinv_of(deg_ref):
    dsum = jnp.sum(deg_ref[...], axis=0, keepdims=True) + 1.0
    dinv_row = lax.rsqrt(dsum)
    one = jnp.ones((1, 1), jnp.float32)
    return lax.dot_general(dinv_row, one, (((0,), (0,)), ((), ())),
                           preferred_element_type=jnp.float32)


def _mm1_body(x_ref, w_ref, deg_ref, out_ref):
    dinv = _dinv_of(deg_ref)
    out_ref[...] = jnp.dot(x_ref[...], w_ref[...],
                           preferred_element_type=jnp.float32) * dinv


def _mm2_body(a0_ref, a1_ref, hp_ref, deg_ref, b_ref, w_ref, out_ref):
    dinv = _dinv_of(deg_ref)
    h1 = (a0_ref[...] + a1_ref[...] + hp_ref[...]) * dinv + b_ref[...]
    h1 = jnp.maximum(h1, 0.0)
    out_ref[...] = jnp.dot(h1, w_ref[...],
                           preferred_element_type=jnp.float32) * dinv


def _make_z_body(dout):
    def _z_body(a0_ref, a1_ref, hp_ref, deg_ref, b_ref,
                zfull_ref, z_ref):
        dinv = _dinv_of(deg_ref)
        zf = (a0_ref[...] + a1_ref[...] + hp_ref[...]) * dinv + b_ref[...]
        zfull_ref[...] = zf
        z_ref[...] = zf[:, :dout]
    return _z_body


def _dec_body(za_ref, zb_ref, out_ref):
    s = lax.dot_general(za_ref[...], zb_ref[...],
                        (((1,), (1,)), ((), ())),
                        preferred_element_type=jnp.float32)
    out_ref[...] = jax.nn.sigmoid(s)


def kernel(x, edge_index, W1, b1, W2, b2):
    n, din = x.shape
    dhid = W1.shape[1]
    dout = W2.shape[1]
    e = edge_index.shape[1]
    ept = e // _NW
    k = _pick_chunk(ept)
    nchunk = ept // k

    n_pad = ((n + 127) // 128) * 128

    src = edge_index[0]
    dst_flat = edge_index[1]
    dst = edge_index[1].reshape(_NW, nchunk, k)
    z1r = jnp.zeros((n_pad,), jnp.float32)
    zhid = jnp.zeros((n_pad, dhid), jnp.float32)
    b1r = b1.reshape(1, dhid)
    b2r = b2.reshape(1, dout)
    d2 = max(dout, 128)
    W2p = jnp.pad(W2, ((0, 0), (0, d2 - dout))) if d2 != dout else W2
    b2p = jnp.pad(b2r, ((0, 0), (0, d2 - dout))) if d2 != dout else b2r

    deg32 = _make_deg(n_pad, ept)(dst_flat, z1r).reshape(_NW, n_pad)

    R = 1024
    grid = (pl.cdiv(n, R),)
    row = lambda i: (i, 0)
    fixed = lambda i: (0, 0)
    dspec = pl.BlockSpec((_NW, R), lambda i: (0, i))

    h1p = pl.pallas_call(
        _mm1_body,
        grid=grid,
        in_specs=[
            pl.BlockSpec((R, din), row),
            pl.BlockSpec((din, dhid), fixed),
            dspec,
        ],
        out_specs=pl.BlockSpec((R, dhid), row),
        out_shape=jax.ShapeDtypeStruct((n, dhid), jnp.float32),
    )(x, W1, deg32)

    agg1 = _make_agg(n_pad, dhid, nchunk, k)(src, dst, h1p, zhid)
    a10, a11 = agg1[0], agg1[1]

    h2p = pl.pallas_call(
        _mm2_body,
        grid=grid,
        in_specs=[
            pl.BlockSpec((R, dhid), row),
            pl.BlockSpec((R, dhid), row),
            pl.BlockSpec((R, dhid), row),
            dspec,
            pl.BlockSpec((1, dhid), fixed),
            pl.BlockSpec((dhid, d2), fixed),
        ],
        out_specs=pl.BlockSpec((R, d2), row),
        out_shape=jax.ShapeDtypeStruct((n, d2), jnp.float32),
    )(a10, a11, h1p, deg32, b1r, W2p)

    zo2 = jnp.zeros((n_pad, d2), jnp.float32)
    agg2 = _make_agg(n_pad, d2, nchunk, k)(src, dst, h2p, zo2)
    a20, a21 = agg2[0], agg2[1]

    zfull, z = pl.pallas_call(
        _make_z_body(dout),
        grid=grid,
        in_specs=[
            pl.BlockSpec((R, d2), row),
            pl.BlockSpec((R, d2), row),
            pl.BlockSpec((R, d2), row),
            dspec,
            pl.BlockSpec((1, d2), fixed),
        ],
        out_specs=[pl.BlockSpec((R, d2), row), pl.BlockSpec((R, dout), row)],
        out_shape=[jax.ShapeDtypeStruct((n, d2), jnp.float32),
                   jax.ShapeDtypeStruct((n, dout), jnp.float32)],
    )(a20, a21, h2p, deg32, b2p)

    BR = 512
    gdec = pl.cdiv(n, BR)
    adj = pl.pallas_call(
        _dec_body,
        grid=(gdec, gdec),
        in_specs=[
            pl.BlockSpec((BR, d2), lambda i, j: (i, 0)),
            pl.BlockSpec((BR, d2), lambda i, j: (j, 0)),
        ],
        out_specs=pl.BlockSpec((BR, BR), lambda i, j: (i, j)),
        out_shape=jax.ShapeDtypeStruct((n, n), jnp.float32),
    )(zfull, zfull)

    return (adj, z)


# speedup vs baseline: 1.5144x; 1.2046x over previous
import functools

import jax
import jax.numpy as jnp
from jax import lax
from jax.experimental import pallas as pl
from jax.experimental.pallas import tpu as pltpu
from jax.experimental.pallas import tpu_sc as plsc

_NC = 2
_NS = 16
_NW = _NC * _NS


def _pick_chunk(ept):
    for k in range(128, 7, -8):
        if ept % k == 0:
            return k
    raise ValueError(f"no valid chunk size for {ept} edges per tile")


def _make_deg(n_pad, ept):
    mesh = plsc.VectorSubcoreMesh(core_axis_name="c", subcore_axis_name="s")
    ngrp = ept // 16

    @functools.partial(
        pl.kernel,
        mesh=mesh,
        out_type=jax.ShapeDtypeStruct((_NW, 1, n_pad), jnp.float32),
        scratch_types=[
            pltpu.VMEM((ept,), jnp.int32),
            pltpu.VMEM((n_pad,), jnp.float32),
        ],
        compiler_params=pltpu.CompilerParams(needs_layout_passes=False),
    )
    def deg(dst_hbm, zero_hbm, out_hbm, dst_v, hist):
        cid = lax.axis_index("c")
        sid = lax.axis_index("s")
        wid = sid * _NC + cid
        pltpu.sync_copy(zero_hbm, hist)
        pltpu.sync_copy(dst_hbm.at[pl.ds(wid * ept, ept)], dst_v)
        ones16 = jnp.ones((16,), jnp.float32)

        def body(g, c):
            idx = dst_v[pl.ds(g * 16, 16)]
            plsc.addupdate_scatter(hist, [idx], ones16)
            return c

        lax.fori_loop(0, ngrp, body, 0)
        pltpu.sync_copy(hist, out_hbm.at[wid, 0])

    return deg


def _make_agg(n_pad, d, nchunk, k):
    rpt = n_pad // _NS
    mesh = plsc.VectorSubcoreMesh(core_axis_name="c", subcore_axis_name="s")

    ept = nchunk * k

    @functools.partial(
        pl.kernel,
        mesh=mesh,
        out_type=jax.ShapeDtypeStruct((_NC, n_pad, d), jnp.float32),
        scratch_types=[
            pltpu.VMEM((nchunk, k), jnp.int32),
            pltpu.VMEM((k,), jnp.int32),
            pltpu.VMEM((k,), jnp.int32),
            pltpu.VMEM((k, d), jnp.float32),
            pltpu.VMEM((k, d), jnp.float32),
            pltpu.VMEM_SHARED((n_pad, d), jnp.float32),
            pltpu.SemaphoreType.DMA,
            pltpu.SemaphoreType.DMA,
            pltpu.SemaphoreType.DMA,
            pltpu.SemaphoreType.DMA,
        ],
    )
    def agg(src_hbm, dst_hbm, h_hbm, zero_hbm, out_hbm,
            dst_v, stage_a, stage_b, rows_a, rows_b, acc,
            sem_a, sem_b, sem_ia, sem_ib):
        cid = lax.axis_index("c")
        sid = lax.axis_index("s")
        wid = sid * _NC + cid
        r0 = sid * rpt
        base = wid * ept
        pltpu.sync_copy(zero_hbm.at[pl.ds(r0, rpt)], acc.at[pl.ds(r0, rpt)])
        pltpu.sync_copy(dst_hbm.at[wid], dst_v)
        plsc.subcore_barrier()

        def sidx(j):
            return src_hbm.at[pl.ds(base + j * k, k)]

        if nchunk % 2 == 1 and nchunk >= 3:
            pltpu.sync_copy(sidx(0), stage_a)
            pltpu.async_copy(h_hbm.at[stage_a], rows_a, sem_a)
            pltpu.async_copy(sidx(1), stage_b, sem_ib)
            last = nchunk - 1

            def body(i, c):
                j0 = 2 * i
                pltpu.make_async_copy(sidx(j0 + 1), stage_b, sem_ib).wait()
                pltpu.make_async_copy(h_hbm.at[stage_a], rows_a, sem_a).wait()
                pltpu.async_copy(h_hbm.at[stage_b], rows_b, sem_b)
                pltpu.async_copy(sidx(j0 + 2), stage_a, sem_ia)
                pltpu.sync_copy(rows_a, acc.at[dst_v.at[j0]], add=True)
                pltpu.make_async_copy(sidx(j0 + 2), stage_a, sem_ia).wait()
                pltpu.make_async_copy(h_hbm.at[stage_b], rows_b, sem_b).wait()
                pltpu.async_copy(h_hbm.at[stage_a], rows_a, sem_a)
                j3 = jnp.minimum(j0 + 3, last)
                pltpu.async_copy(sidx(j3), stage_b, sem_ib)
                pltpu.sync_copy(rows_b, acc.at[dst_v.at[j0 + 1]], add=True)
                return c

            lax.fori_loop(0, (nchunk - 1) // 2, body, 0)
            pltpu.make_async_copy(h_hbm.at[stage_a], rows_a, sem_a).wait()
            pltpu.make_async_copy(sidx(last), stage_b, sem_ib).wait()
            pltpu.sync_copy(rows_a, acc.at[dst_v.at[last]], add=True)
        else:
            def body(j, c):
                pltpu.sync_copy(sidx(j), stage_a)
                pltpu.async_copy(h_hbm.at[stage_a], rows_a, sem_a).wait()
                pltpu.sync_copy(rows_a, acc.at[dst_v.at[j]], add=True)
                return c

            lax.fori_loop(0, nchunk, body, 0)

        plsc.subcore_barrier()
        pltpu.sync_copy(acc.at[pl.ds(r0, rpt)], out_hbm.at[cid, pl.ds(r0, rpt)])

    return agg


def _dinv_of(deg_ref):
    dsum = jnp.sum(deg_ref[...], axis=0, keepdims=True) + 1.0
    dinv_row = lax.rsqrt(dsum)
    one = jnp.ones((1, 1), jnp.float32)
    return lax.dot_general(dinv_row, one, (((0,), (0,)), ((), ())),
                           preferred_element_type=jnp.float32)


def _mm1_body(x_ref, w_ref, deg_ref, out_ref):
    dinv = _dinv_of(deg_ref)
    out_ref[...] = jnp.dot(x_ref[...], w_ref[...],
                           preferred_element_type=jnp.float32) * dinv


def _mm2_body(a0_ref, a1_ref, hp_ref, deg_ref, b_ref, w_ref, out_ref):
    dinv = _dinv_of(deg_ref)
    h1 = (a0_ref[...] + a1_ref[...] + hp_ref[...]) * dinv + b_ref[...]
    h1 = jnp.maximum(h1, 0.0)
    out_ref[...] = jnp.dot(h1, w_ref[...],
                           preferred_element_type=jnp.float32) * dinv


def _make_z_body(dout):
    def _z_body(a0_ref, a1_ref, hp_ref, deg_ref, b_ref,
                zfull_ref, z_ref):
        dinv = _dinv_of(deg_ref)
        zf = (a0_ref[...] + a1_ref[...] + hp_ref[...]) * dinv + b_ref[...]
        zfull_ref[...] = zf
        z_ref[...] = zf[:, :dout]
    return _z_body


def _dec_body(za_ref, zb_ref, out_ref):
    s = lax.dot_general(za_ref[...], zb_ref[...],
                        (((1,), (1,)), ((), ())),
                        preferred_element_type=jnp.float32)
    out_ref[...] = 0.5 * jnp.tanh(0.5 * s) + 0.5


def kernel(x, edge_index, W1, b1, W2, b2):
    n, din = x.shape
    dhid = W1.shape[1]
    dout = W2.shape[1]
    e = edge_index.shape[1]
    ept = e // _NW
    k = _pick_chunk(ept)
    nchunk = ept // k

    n_pad = ((n + 127) // 128) * 128

    src = edge_index[0]
    dst_flat = edge_index[1]
    dst = edge_index[1].reshape(_NW, nchunk, k)
    z1r = jnp.zeros((n_pad,), jnp.float32)
    zhid = jnp.zeros((n_pad, dhid), jnp.float32)
    b1r = b1.reshape(1, dhid)
    b2r = b2.reshape(1, dout)
    d2 = max(dout, 128)
    W2p = jnp.pad(W2, ((0, 0), (0, d2 - dout))) if d2 != dout else W2
    b2p = jnp.pad(b2r, ((0, 0), (0, d2 - dout))) if d2 != dout else b2r

    deg32 = _make_deg(n_pad, ept)(dst_flat, z1r).reshape(_NW, n_pad)

    R = 1024
    grid = (pl.cdiv(n, R),)
    row = lambda i: (i, 0)
    fixed = lambda i: (0, 0)
    dspec = pl.BlockSpec((_NW, R), lambda i: (0, i))

    h1p = pl.pallas_call(
        _mm1_body,
        grid=grid,
        in_specs=[
            pl.BlockSpec((R, din), row),
            pl.BlockSpec((din, dhid), fixed),
            dspec,
        ],
        out_specs=pl.BlockSpec((R, dhid), row),
        out_shape=jax.ShapeDtypeStruct((n, dhid), jnp.float32),
    )(x, W1, deg32)

    agg1 = _make_agg(n_pad, dhid, nchunk, k)(src, dst, h1p, zhid)
    a10, a11 = agg1[0], agg1[1]

    h2p = pl.pallas_call(
        _mm2_body,
        grid=grid,
        in_specs=[
            pl.BlockSpec((R, dhid), row),
            pl.BlockSpec((R, dhid), row),
            pl.BlockSpec((R, dhid), row),
            dspec,
            pl.BlockSpec((1, dhid), fixed),
            pl.BlockSpec((dhid, d2), fixed),
        ],
        out_specs=pl.BlockSpec((R, d2), row),
        out_shape=jax.ShapeDtypeStruct((n, d2), jnp.float32),
    )(a10, a11, h1p, deg32, b1r, W2p)

    zo2 = jnp.zeros((n_pad, d2), jnp.float32)
    agg2 = _make_agg(n_pad, d2, nchunk, k)(src, dst, h2p, zo2)
    a20, a21 = agg2[0], agg2[1]

    zfull, z = pl.pallas_call(
        _make_z_body(dout),
        grid=grid,
        in_specs=[
            pl.BlockSpec((R, d2), row),
            pl.BlockSpec((R, d2), row),
            pl.BlockSpec((R, d2), row),
            dspec,
            pl.BlockSpec((1, d2), fixed),
        ],
        out_specs=[pl.BlockSpec((R, d2), row), pl.BlockSpec((R, dout), row)],
        out_shape=[jax.ShapeDtypeStruct((n, d2), jnp.float32),
                   jax.ShapeDtypeStruct((n, dout), jnp.float32)],
    )(a20, a21, h2p, deg32, b2p)

    BRI, BRJ = 512, 1024
    adj = pl.pallas_call(
        _dec_body,
        grid=(pl.cdiv(n, BRI), pl.cdiv(n, BRJ)),
        in_specs=[
            pl.BlockSpec((BRI, d2), lambda i, j: (i, 0)),
            pl.BlockSpec((BRJ, d2), lambda i, j: (j, 0)),
        ],
        out_specs=pl.BlockSpec((BRI, BRJ), lambda i, j: (i, j)),
        out_shape=jax.ShapeDtypeStruct((n, n), jnp.float32),
    )(zfull, zfull)

    return (adj, z)


# speedup vs baseline: 1.8217x; 1.2029x over previous
import functools

import jax
import jax.numpy as jnp
from jax import lax
from jax.experimental import pallas as pl
from jax.experimental.pallas import tpu as pltpu
from jax.experimental.pallas import tpu_sc as plsc

_NC = 2
_NS = 16
_NW = _NC * _NS


def _pick_chunk(ept):
    for k in range(128, 7, -8):
        if ept % k == 0:
            return k
    raise ValueError(f"no valid chunk size for {ept} edges per tile")


def _make_deg(n_pad, ept):
    mesh = plsc.VectorSubcoreMesh(core_axis_name="c", subcore_axis_name="s")
    ngrp = ept // 16

    @functools.partial(
        pl.kernel,
        mesh=mesh,
        out_type=jax.ShapeDtypeStruct((_NW, 1, n_pad), jnp.float32),
        scratch_types=[
            pltpu.VMEM((ept,), jnp.int32),
            pltpu.VMEM((n_pad,), jnp.float32),
        ],
        compiler_params=pltpu.CompilerParams(needs_layout_passes=False),
    )
    def deg(dst_hbm, zero_hbm, out_hbm, dst_v, hist):
        cid = lax.axis_index("c")
        sid = lax.axis_index("s")
        wid = sid * _NC + cid
        pltpu.sync_copy(zero_hbm, hist)
        pltpu.sync_copy(dst_hbm.at[pl.ds(wid * ept, ept)], dst_v)
        ones16 = jnp.ones((16,), jnp.float32)

        def body(g, c):
            idx = dst_v[pl.ds(g * 16, 16)]
            plsc.addupdate_scatter(hist, [idx], ones16)
            return c

        lax.fori_loop(0, ngrp, body, 0)
        pltpu.sync_copy(hist, out_hbm.at[wid, 0])

    return deg


def _make_agg(n_pad, d, nchunk, k):
    rpt = n_pad // _NS
    mesh = plsc.VectorSubcoreMesh(core_axis_name="c", subcore_axis_name="s")

    ept = nchunk * k

    @functools.partial(
        pl.kernel,
        mesh=mesh,
        out_type=jax.ShapeDtypeStruct((_NC, n_pad, d), jnp.float32),
        scratch_types=[
            pltpu.VMEM((nchunk, k), jnp.int32),
            pltpu.VMEM((k,), jnp.int32),
            pltpu.VMEM((k,), jnp.int32),
            pltpu.VMEM((k,), jnp.int32),
            pltpu.VMEM((k, d), jnp.float32),
            pltpu.VMEM((k, d), jnp.float32),
            pltpu.VMEM((k, d), jnp.float32),
            pltpu.VMEM_SHARED((n_pad, d), jnp.float32),
            pltpu.SemaphoreType.DMA,
            pltpu.SemaphoreType.DMA,
            pltpu.SemaphoreType.DMA,
            pltpu.SemaphoreType.DMA,
            pltpu.SemaphoreType.DMA,
            pltpu.SemaphoreType.DMA,
        ],
    )
    def agg(src_hbm, dst_hbm, h_hbm, zero_hbm, out_hbm,
            dst_v, stage_a, stage_b, stage_c, rows_a, rows_b, rows_c, acc,
            sem_a, sem_b, sem_c, sem_ia, sem_ib, sem_ic):
        cid = lax.axis_index("c")
        sid = lax.axis_index("s")
        wid = sid * _NC + cid
        r0 = sid * rpt
        base = wid * ept
        pltpu.sync_copy(zero_hbm.at[pl.ds(r0, rpt)], acc.at[pl.ds(r0, rpt)])
        pltpu.sync_copy(dst_hbm.at[wid], dst_v)
        plsc.subcore_barrier()

        def sidx(j):
            return src_hbm.at[pl.ds(base + j * k, k)]

        last = nchunk - 1
        if nchunk % 3 == 2 and nchunk >= 5:
            pltpu.sync_copy(sidx(0), stage_a)
            pltpu.sync_copy(sidx(1), stage_b)
            pltpu.async_copy(h_hbm.at[stage_a], rows_a, sem_a)
            pltpu.async_copy(h_hbm.at[stage_b], rows_b, sem_b)
            pltpu.async_copy(sidx(2), stage_c, sem_ic)

            def body(i, c):
                j0 = 3 * i
                pltpu.make_async_copy(sidx(j0 + 2), stage_c, sem_ic).wait()
                pltpu.make_async_copy(h_hbm.at[stage_a], rows_a, sem_a).wait()
                pltpu.async_copy(h_hbm.at[stage_c], rows_c, sem_c)
                pltpu.async_copy(sidx(j0 + 3), stage_a, sem_ia)
                pltpu.sync_copy(rows_a, acc.at[dst_v.at[j0]], add=True)
                pltpu.make_async_copy(sidx(j0 + 3), stage_a, sem_ia).wait()
                pltpu.make_async_copy(h_hbm.at[stage_b], rows_b, sem_b).wait()
                pltpu.async_copy(h_hbm.at[stage_a], rows_a, sem_a)
                pltpu.async_copy(sidx(j0 + 4), stage_b, sem_ib)
                pltpu.sync_copy(rows_b, acc.at[dst_v.at[j0 + 1]], add=True)
                pltpu.make_async_copy(sidx(j0 + 4), stage_b, sem_ib).wait()
                pltpu.make_async_copy(h_hbm.at[stage_c], rows_c, sem_c).wait()
                pltpu.async_copy(h_hbm.at[stage_b], rows_b, sem_b)
                j5 = jnp.minimum(j0 + 5, last)
                pltpu.async_copy(sidx(j5), stage_c, sem_ic)
                pltpu.sync_copy(rows_c, acc.at[dst_v.at[j0 + 2]], add=True)
                return c

            lax.fori_loop(0, (nchunk - 2) // 3, body, 0)
            pltpu.make_async_copy(h_hbm.at[stage_a], rows_a, sem_a).wait()
            pltpu.sync_copy(rows_a, acc.at[dst_v.at[last - 1]], add=True)
            pltpu.make_async_copy(h_hbm.at[stage_b], rows_b, sem_b).wait()
            pltpu.make_async_copy(sidx(last), stage_c, sem_ic).wait()
            pltpu.sync_copy(rows_b, acc.at[dst_v.at[last]], add=True)
        elif nchunk % 2 == 1 and nchunk >= 3:
            pltpu.sync_copy(sidx(0), stage_a)
            pltpu.async_copy(h_hbm.at[stage_a], rows_a, sem_a)
            pltpu.async_copy(sidx(1), stage_b, sem_ib)

            def body(i, c):
                j0 = 2 * i
                pltpu.make_async_copy(sidx(j0 + 1), stage_b, sem_ib).wait()
                pltpu.make_async_copy(h_hbm.at[stage_a], rows_a, sem_a).wait()
                pltpu.async_copy(h_hbm.at[stage_b], rows_b, sem_b)
                pltpu.async_copy(sidx(j0 + 2), stage_a, sem_ia)
                pltpu.sync_copy(rows_a, acc.at[dst_v.at[j0]], add=True)
                pltpu.make_async_copy(sidx(j0 + 2), stage_a, sem_ia).wait()
                pltpu.make_async_copy(h_hbm.at[stage_b], rows_b, sem_b).wait()
                pltpu.async_copy(h_hbm.at[stage_a], rows_a, sem_a)
                j3 = jnp.minimum(j0 + 3, last)
                pltpu.async_copy(sidx(j3), stage_b, sem_ib)
                pltpu.sync_copy(rows_b, acc.at[dst_v.at[j0 + 1]], add=True)
                return c

            lax.fori_loop(0, (nchunk - 1) // 2, body, 0)
            pltpu.make_async_copy(h_hbm.at[stage_a], rows_a, sem_a).wait()
            pltpu.make_async_copy(sidx(last), stage_b, sem_ib).wait()
            pltpu.sync_copy(rows_a, acc.at[dst_v.at[last]], add=True)
        else:
            def body(j, c):
                pltpu.sync_copy(sidx(j), stage_a)
                pltpu.async_copy(h_hbm.at[stage_a], rows_a, sem_a).wait()
                pltpu.sync_copy(rows_a, acc.at[dst_v.at[j]], add=True)
                return c

            lax.fori_loop(0, nchunk, body, 0)

        plsc.subcore_barrier()
        pltpu.sync_copy(acc.at[pl.ds(r0, rpt)], out_hbm.at[cid, pl.ds(r0, rpt)])

    return agg


def _dinv_of(deg_ref):
    dsum = jnp.sum(deg_ref[...], axis=0, keepdims=True) + 1.0
    dinv_row = lax.rsqrt(dsum)
    one = jnp.ones((1, 1), jnp.float32)
    return lax.dot_general(dinv_row, one, (((0,), (0,)), ((), ())),
                           preferred_element_type=jnp.float32)


def _mm1_body(x_ref, w_ref, deg_ref, out_ref):
    dinv = _dinv_of(deg_ref)
    out_ref[...] = jnp.dot(x_ref[...], w_ref[...],
                           preferred_element_type=jnp.float32) * dinv


def _mm2_body(a0_ref, a1_ref, hp_ref, deg_ref, b_ref, w_ref, out_ref):
    dinv = _dinv_of(deg_ref)
    h1 = (a0_ref[...] + a1_ref[...] + hp_ref[...]) * dinv + b_ref[...]
    h1 = jnp.maximum(h1, 0.0)
    out_ref[...] = jnp.dot(h1, w_ref[...],
                           preferred_element_type=jnp.float32) * dinv


def _make_z_body(dout):
    def _z_body(a0_ref, a1_ref, hp_ref, deg_ref, b_ref,
                zfull_ref, z_ref):
        dinv = _dinv_of(deg_ref)
        zf = (a0_ref[...] + a1_ref[...] + hp_ref[...]) * dinv + b_ref[...]
        zfull_ref[...] = zf
        z_ref[...] = zf[:, :dout]
    return _z_body


def _dec_body(za_ref, zb_ref, out_ref):
    s = lax.dot_general(za_ref[...], zb_ref[...],
                        (((1,), (1,)), ((), ())),
                        preferred_element_type=jnp.float32)
    out_ref[...] = 0.5 * jnp.tanh(0.5 * s) + 0.5


def kernel(x, edge_index, W1, b1, W2, b2):
    n, din = x.shape
    dhid = W1.shape[1]
    dout = W2.shape[1]
    e = edge_index.shape[1]
    ept = e // _NW
    k = _pick_chunk(ept)
    nchunk = ept // k

    n_pad = ((n + 127) // 128) * 128

    src = edge_index[0]
    dst_flat = edge_index[1]
    dst = edge_index[1].reshape(_NW, nchunk, k)
    z1r = jnp.zeros((n_pad,), jnp.float32)
    zhid = jnp.zeros((n_pad, dhid), jnp.float32)
    b1r = b1.reshape(1, dhid)
    b2r = b2.reshape(1, dout)
    d2 = max(dout, 128)
    W2p = jnp.pad(W2, ((0, 0), (0, d2 - dout))) if d2 != dout else W2
    b2p = jnp.pad(b2r, ((0, 0), (0, d2 - dout))) if d2 != dout else b2r

    deg32 = _make_deg(n_pad, ept)(dst_flat, z1r).reshape(_NW, n_pad)

    R = 1024
    grid = (pl.cdiv(n, R),)
    row = lambda i: (i, 0)
    fixed = lambda i: (0, 0)
    dspec = pl.BlockSpec((_NW, R), lambda i: (0, i))

    h1p = pl.pallas_call(
        _mm1_body,
        grid=grid,
        in_specs=[
            pl.BlockSpec((R, din), row),
            pl.BlockSpec((din, dhid), fixed),
            dspec,
        ],
        out_specs=pl.BlockSpec((R, dhid), row),
        out_shape=jax.ShapeDtypeStruct((n, dhid), jnp.float32),
    )(x, W1, deg32)

    agg1 = _make_agg(n_pad, dhid, nchunk, k)(src, dst, h1p, zhid)
    a10, a11 = agg1[0], agg1[1]

    h2p = pl.pallas_call(
        _mm2_body,
        grid=grid,
        in_specs=[
            pl.BlockSpec((R, dhid), row),
            pl.BlockSpec((R, dhid), row),
            pl.BlockSpec((R, dhid), row),
            dspec,
            pl.BlockSpec((1, dhid), fixed),
            pl.BlockSpec((dhid, d2), fixed),
        ],
        out_specs=pl.BlockSpec((R, d2), row),
        out_shape=jax.ShapeDtypeStruct((n, d2), jnp.float32),
    )(a10, a11, h1p, deg32, b1r, W2p)

    zo2 = jnp.zeros((n_pad, d2), jnp.float32)
    agg2 = _make_agg(n_pad, d2, nchunk, k)(src, dst, h2p, zo2)
    a20, a21 = agg2[0], agg2[1]

    zfull, z = pl.pallas_call(
        _make_z_body(dout),
        grid=grid,
        in_specs=[
            pl.BlockSpec((R, d2), row),
            pl.BlockSpec((R, d2), row),
            pl.BlockSpec((R, d2), row),
            dspec,
            pl.BlockSpec((1, d2), fixed),
        ],
        out_specs=[pl.BlockSpec((R, d2), row), pl.BlockSpec((R, dout), row)],
        out_shape=[jax.ShapeDtypeStruct((n, d2), jnp.float32),
                   jax.ShapeDtypeStruct((n, dout), jnp.float32)],
    )(a20, a21, h2p, deg32, b2p)

    BRI, BRJ = 512, 1024
    adj = pl.pallas_call(
        _dec_body,
        grid=(pl.cdiv(n, BRI), pl.cdiv(n, BRJ)),
        in_specs=[
            pl.BlockSpec((BRI, d2), lambda i, j: (i, 0)),
            pl.BlockSpec((BRJ, d2), lambda i, j: (j, 0)),
        ],
        out_specs=pl.BlockSpec((BRI, BRJ), lambda i, j: (i, j)),
        out_shape=jax.ShapeDtypeStruct((n, n), jnp.float32),
    )(zfull, zfull)

    return (adj, z)


# speedup vs baseline: 2.0425x; 1.1212x over previous
import functools

import jax
import jax.numpy as jnp
from jax import lax
from jax.experimental import pallas as pl
from jax.experimental.pallas import tpu as pltpu
from jax.experimental.pallas import tpu_sc as plsc

_NC = 2
_NS = 16
_NW = _NC * _NS


def _pick_chunk(ept):
    for k in range(128, 7, -8):
        if ept % k == 0:
            return k
    raise ValueError(f"no valid chunk size for {ept} edges per tile")


def _make_deg(n_pad, ept):
    mesh = plsc.VectorSubcoreMesh(core_axis_name="c", subcore_axis_name="s")
    ngrp = ept // 16

    @functools.partial(
        pl.kernel,
        mesh=mesh,
        out_type=jax.ShapeDtypeStruct((_NW, 1, n_pad), jnp.float32),
        scratch_types=[
            pltpu.VMEM((ept,), jnp.int32),
            pltpu.VMEM((n_pad,), jnp.float32),
        ],
        compiler_params=pltpu.CompilerParams(needs_layout_passes=False),
    )
    def deg(dst_hbm, zero_hbm, out_hbm, dst_v, hist):
        cid = lax.axis_index("c")
        sid = lax.axis_index("s")
        wid = sid * _NC + cid
        pltpu.sync_copy(zero_hbm, hist)
        pltpu.sync_copy(dst_hbm.at[pl.ds(wid * ept, ept)], dst_v)
        ones16 = jnp.ones((16,), jnp.float32)

        def body(g, c):
            idx = dst_v[pl.ds(g * 16, 16)]
            plsc.addupdate_scatter(hist, [idx], ones16)
            return c

        lax.fori_loop(0, ngrp, body, 0)
        pltpu.sync_copy(hist, out_hbm.at[wid, 0])

    return deg


def _make_agg(n_pad, d, nchunk, k):
    rpt = n_pad // _NS
    mesh = plsc.VectorSubcoreMesh(core_axis_name="c", subcore_axis_name="s")

    ept = nchunk * k

    @functools.partial(
        pl.kernel,
        mesh=mesh,
        out_type=jax.ShapeDtypeStruct((_NC, n_pad, d), jnp.float32),
        scratch_types=[
            pltpu.VMEM((nchunk, k), jnp.int32),
            pltpu.VMEM((k,), jnp.int32),
            pltpu.VMEM((k,), jnp.int32),
            pltpu.VMEM((k,), jnp.int32),
            pltpu.VMEM((k, d), jnp.float32),
            pltpu.VMEM((k, d), jnp.float32),
            pltpu.VMEM((k, d), jnp.float32),
            pltpu.VMEM_SHARED((n_pad, d), jnp.float32),
            pltpu.SemaphoreType.DMA,
            pltpu.SemaphoreType.DMA,
            pltpu.SemaphoreType.DMA,
            pltpu.SemaphoreType.DMA,
            pltpu.SemaphoreType.DMA,
            pltpu.SemaphoreType.DMA,
        ],
    )
    def agg(src_hbm, dst_hbm, h_hbm, zero_hbm, out_hbm,
            dst_v, stage_a, stage_b, stage_c, rows_a, rows_b, rows_c, acc,
            sem_a, sem_b, sem_c, sem_ia, sem_ib, sem_ic):
        cid = lax.axis_index("c")
        sid = lax.axis_index("s")
        wid = sid * _NC + cid
        r0 = sid * rpt
        base = wid * ept
        pltpu.sync_copy(zero_hbm.at[pl.ds(r0, rpt)], acc.at[pl.ds(r0, rpt)])
        pltpu.sync_copy(dst_hbm.at[wid], dst_v)
        plsc.subcore_barrier()

        def sidx(j):
            return src_hbm.at[pl.ds(base + j * k, k)]

        last = nchunk - 1
        if nchunk % 3 == 2 and nchunk >= 5:
            pltpu.sync_copy(sidx(0), stage_a)
            pltpu.sync_copy(sidx(1), stage_b)
            pltpu.async_copy(h_hbm.at[stage_a], rows_a, sem_a)
            pltpu.async_copy(h_hbm.at[stage_b], rows_b, sem_b)
            pltpu.async_copy(sidx(2), stage_c, sem_ic)

            def body(i, c):
                j0 = 3 * i
                pltpu.make_async_copy(sidx(j0 + 2), stage_c, sem_ic).wait()
                pltpu.make_async_copy(h_hbm.at[stage_a], rows_a, sem_a).wait()
                pltpu.async_copy(h_hbm.at[stage_c], rows_c, sem_c)
                pltpu.async_copy(sidx(j0 + 3), stage_a, sem_ia)
                pltpu.sync_copy(rows_a, acc.at[dst_v.at[j0]], add=True)
                pltpu.make_async_copy(sidx(j0 + 3), stage_a, sem_ia).wait()
                pltpu.make_async_copy(h_hbm.at[stage_b], rows_b, sem_b).wait()
                pltpu.async_copy(h_hbm.at[stage_a], rows_a, sem_a)
                pltpu.async_copy(sidx(j0 + 4), stage_b, sem_ib)
                pltpu.sync_copy(rows_b, acc.at[dst_v.at[j0 + 1]], add=True)
                pltpu.make_async_copy(sidx(j0 + 4), stage_b, sem_ib).wait()
                pltpu.make_async_copy(h_hbm.at[stage_c], rows_c, sem_c).wait()
                pltpu.async_copy(h_hbm.at[stage_b], rows_b, sem_b)
                j5 = jnp.minimum(j0 + 5, last)
                pltpu.async_copy(sidx(j5), stage_c, sem_ic)
                pltpu.sync_copy(rows_c, acc.at[dst_v.at[j0 + 2]], add=True)
                return c

            lax.fori_loop(0, (nchunk - 2) // 3, body, 0)
            pltpu.make_async_copy(h_hbm.at[stage_a], rows_a, sem_a).wait()
            pltpu.sync_copy(rows_a, acc.at[dst_v.at[last - 1]], add=True)
            pltpu.make_async_copy(h_hbm.at[stage_b], rows_b, sem_b).wait()
            pltpu.make_async_copy(sidx(last), stage_c, sem_ic).wait()
            pltpu.sync_copy(rows_b, acc.at[dst_v.at[last]], add=True)
        elif nchunk % 2 == 1 and nchunk >= 3:
            pltpu.sync_copy(sidx(0), stage_a)
            pltpu.async_copy(h_hbm.at[stage_a], rows_a, sem_a)
            pltpu.async_copy(sidx(1), stage_b, sem_ib)

            def body(i, c):
                j0 = 2 * i
                pltpu.make_async_copy(sidx(j0 + 1), stage_b, sem_ib).wait()
                pltpu.make_async_copy(h_hbm.at[stage_a], rows_a, sem_a).wait()
                pltpu.async_copy(h_hbm.at[stage_b], rows_b, sem_b)
                pltpu.async_copy(sidx(j0 + 2), stage_a, sem_ia)
                pltpu.sync_copy(rows_a, acc.at[dst_v.at[j0]], add=True)
                pltpu.make_async_copy(sidx(j0 + 2), stage_a, sem_ia).wait()
                pltpu.make_async_copy(h_hbm.at[stage_b], rows_b, sem_b).wait()
                pltpu.async_copy(h_hbm.at[stage_a], rows_a, sem_a)
                j3 = jnp.minimum(j0 + 3, last)
                pltpu.async_copy(sidx(j3), stage_b, sem_ib)
                pltpu.sync_copy(rows_b, acc.at[dst_v.at[j0 + 1]], add=True)
                return c

            lax.fori_loop(0, (nchunk - 1) // 2, body, 0)
            pltpu.make_async_copy(h_hbm.at[stage_a], rows_a, sem_a).wait()
            pltpu.make_async_copy(sidx(last), stage_b, sem_ib).wait()
            pltpu.sync_copy(rows_a, acc.at[dst_v.at[last]], add=True)
        else:
            def body(j, c):
                pltpu.sync_copy(sidx(j), stage_a)
                pltpu.async_copy(h_hbm.at[stage_a], rows_a, sem_a).wait()
                pltpu.sync_copy(rows_a, acc.at[dst_v.at[j]], add=True)
                return c

            lax.fori_loop(0, nchunk, body, 0)

        plsc.subcore_barrier()
        pltpu.sync_copy(acc.at[pl.ds(r0, rpt)], out_hbm.at[cid, pl.ds(r0, rpt)])

    return agg


def _dinv_of(deg_ref):
    dsum = jnp.sum(deg_ref[...], axis=0, keepdims=True) + 1.0
    dinv_row = lax.rsqrt(dsum)
    one = jnp.ones((1, 1), jnp.float32)
    return lax.dot_general(dinv_row, one, (((0,), (0,)), ((), ())),
                           preferred_element_type=jnp.float32)


def _mm1_body(x_ref, w_ref, deg_ref, out_ref):
    dinv = _dinv_of(deg_ref)
    out_ref[...] = jnp.dot(x_ref[...], w_ref[...],
                           preferred_element_type=jnp.float32) * dinv


def _mm2_body(a0_ref, a1_ref, hp_ref, deg_ref, b_ref, w_ref, out_ref):
    dinv = _dinv_of(deg_ref)
    h1 = (a0_ref[...] + a1_ref[...] + hp_ref[...]) * dinv + b_ref[...]
    h1 = jnp.maximum(h1, 0.0)
    out_ref[...] = jnp.dot(h1, w_ref[...],
                           preferred_element_type=jnp.float32) * dinv


def _make_z_body(dout):
    def _z_body(a0_ref, a1_ref, hp_ref, deg_ref, b_ref,
                zfull_ref, z_ref):
        dinv = _dinv_of(deg_ref)
        zf = (a0_ref[...] + a1_ref[...] + hp_ref[...]) * dinv + b_ref[...]
        zfull_ref[...] = zf
        z_ref[...] = zf[:, :dout]
    return _z_body


def _dec_body(za_ref, zb_ref, out_ref):
    s = lax.dot_general(za_ref[...], zb_ref[...],
                        (((1,), (1,)), ((), ())),
                        preferred_element_type=jnp.float32)
    out_ref[...] = 0.5 * jnp.tanh(0.5 * s) + 0.5


def kernel(x, edge_index, W1, b1, W2, b2):
    n, din = x.shape
    dhid = W1.shape[1]
    dout = W2.shape[1]
    e = edge_index.shape[1]
    ept = e // _NW
    k = _pick_chunk(ept)
    nchunk = ept // k

    n_pad = ((n + 127) // 128) * 128

    src = edge_index[0]
    dst_flat = edge_index[1]
    dst = edge_index[1].reshape(_NW, nchunk, k)
    z1r = jnp.zeros((n_pad,), jnp.float32)
    zhid = jnp.zeros((n_pad, dhid), jnp.float32)
    b1r = b1.reshape(1, dhid)
    b2r = b2.reshape(1, dout)
    d2 = max(dout, 128)
    W2p = jnp.pad(W2, ((0, 0), (0, d2 - dout))) if d2 != dout else W2
    b2p = jnp.pad(b2r, ((0, 0), (0, d2 - dout))) if d2 != dout else b2r

    deg32 = _make_deg(n_pad, ept)(dst_flat, z1r).reshape(_NW, n_pad)

    R = 1024
    grid = (pl.cdiv(n, R),)
    row = lambda i: (i, 0)
    fixed = lambda i: (0, 0)
    dspec = pl.BlockSpec((_NW, R), lambda i: (0, i))

    h1p = pl.pallas_call(
        _mm1_body,
        grid=grid,
        in_specs=[
            pl.BlockSpec((R, din), row),
            pl.BlockSpec((din, dhid), fixed),
            dspec,
        ],
        out_specs=pl.BlockSpec((R, dhid), row),
        out_shape=jax.ShapeDtypeStruct((n, dhid), jnp.float32),
    )(x, W1, deg32)

    agg1 = _make_agg(n_pad, dhid, nchunk, k)(src, dst, h1p, zhid)
    a10, a11 = agg1[0], agg1[1]

    h2p = pl.pallas_call(
        _mm2_body,
        grid=grid,
        in_specs=[
            pl.BlockSpec((R, dhid), row),
            pl.BlockSpec((R, dhid), row),
            pl.BlockSpec((R, dhid), row),
            dspec,
            pl.BlockSpec((1, dhid), fixed),
            pl.BlockSpec((dhid, d2), fixed),
        ],
        out_specs=pl.BlockSpec((R, d2), row),
        out_shape=jax.ShapeDtypeStruct((n, d2), jnp.float32),
    )(a10, a11, h1p, deg32, b1r, W2p)

    zo2 = jnp.zeros((n_pad, d2), jnp.float32)
    agg2 = _make_agg(n_pad, d2, nchunk, k)(src, dst, h2p, zo2)
    a20, a21 = agg2[0], agg2[1]

    zfull, z = pl.pallas_call(
        _make_z_body(dout),
        grid=grid,
        in_specs=[
            pl.BlockSpec((R, d2), row),
            pl.BlockSpec((R, d2), row),
            pl.BlockSpec((R, d2), row),
            dspec,
            pl.BlockSpec((1, d2), fixed),
        ],
        out_specs=[pl.BlockSpec((R, d2), row), pl.BlockSpec((R, dout), row)],
        out_shape=[jax.ShapeDtypeStruct((n, d2), jnp.float32),
                   jax.ShapeDtypeStruct((n, dout), jnp.float32)],
    )(a20, a21, h2p, deg32, b2p)

    BRI, BRJ = 512, 2048
    adj = pl.pallas_call(
        _dec_body,
        grid=(pl.cdiv(n, BRI), pl.cdiv(n, BRJ)),
        in_specs=[
            pl.BlockSpec((BRI, d2), lambda i, j: (i, 0)),
            pl.BlockSpec((BRJ, d2), lambda i, j: (j, 0)),
        ],
        out_specs=pl.BlockSpec((BRI, BRJ), lambda i, j: (i, j)),
        out_shape=jax.ShapeDtypeStruct((n, n), jnp.float32),
    )(zfull, zfull)

    return (adj, z)


# speedup vs baseline: 2.2160x; 1.0850x over previous
import functools

import jax
import jax.numpy as jnp
from jax import lax
from jax.experimental import pallas as pl
from jax.experimental.pallas import tpu as pltpu
from jax.experimental.pallas import tpu_sc as plsc

_NC = 2
_NS = 16
_NW = _NC * _NS


def _pick_chunk(ept):
    for k in range(128, 7, -8):
        if ept % k == 0:
            return k
    raise ValueError(f"no valid chunk size for {ept} edges per tile")


def _make_deg(n_pad, ept):
    mesh = plsc.VectorSubcoreMesh(core_axis_name="c", subcore_axis_name="s")
    ngrp = ept // 16

    @functools.partial(
        pl.kernel,
        mesh=mesh,
        out_type=jax.ShapeDtypeStruct((_NW, 1, n_pad), jnp.float32),
        scratch_types=[
            pltpu.VMEM((ept,), jnp.int32),
            pltpu.VMEM((n_pad,), jnp.float32),
        ],
        compiler_params=pltpu.CompilerParams(needs_layout_passes=False),
    )
    def deg(dst_hbm, zero_hbm, out_hbm, dst_v, hist):
        cid = lax.axis_index("c")
        sid = lax.axis_index("s")
        wid = sid * _NC + cid
        pltpu.sync_copy(zero_hbm, hist)
        pltpu.sync_copy(dst_hbm.at[pl.ds(wid * ept, ept)], dst_v)
        ones16 = jnp.ones((16,), jnp.float32)

        def body(g, c):
            idx = dst_v[pl.ds(g * 16, 16)]
            plsc.addupdate_scatter(hist, [idx], ones16)
            return c

        lax.fori_loop(0, ngrp, body, 0)
        pltpu.sync_copy(hist, out_hbm.at[wid, 0])

    return deg


def _make_agg(n_pad, d, nchunk, k):
    rpt = n_pad // _NS
    mesh = plsc.VectorSubcoreMesh(core_axis_name="c", subcore_axis_name="s")

    ept = nchunk * k

    @functools.partial(
        pl.kernel,
        mesh=mesh,
        out_type=jax.ShapeDtypeStruct((_NC, n_pad, d), jnp.float32),
        scratch_types=[
            pltpu.VMEM((nchunk, k), jnp.int32),
            pltpu.VMEM((k,), jnp.int32),
            pltpu.VMEM((k,), jnp.int32),
            pltpu.VMEM((k,), jnp.int32),
            pltpu.VMEM((k, d), jnp.float32),
            pltpu.VMEM((k, d), jnp.float32),
            pltpu.VMEM((k, d), jnp.float32),
            pltpu.VMEM_SHARED((n_pad, d), jnp.float32),
            pltpu.SemaphoreType.DMA,
            pltpu.SemaphoreType.DMA,
            pltpu.SemaphoreType.DMA,
            pltpu.SemaphoreType.DMA,
            pltpu.SemaphoreType.DMA,
            pltpu.SemaphoreType.DMA,
        ],
    )
    def agg(src_hbm, dst_hbm, h_hbm, zero_hbm, out_hbm,
            dst_v, stage_a, stage_b, stage_c, rows_a, rows_b, rows_c, acc,
            sem_a, sem_b, sem_c, sem_ia, sem_ib, sem_ic):
        cid = lax.axis_index("c")
        sid = lax.axis_index("s")
        wid = sid * _NC + cid
        r0 = sid * rpt
        base = wid * ept
        pltpu.sync_copy(zero_hbm.at[pl.ds(r0, rpt)], acc.at[pl.ds(r0, rpt)])
        pltpu.sync_copy(dst_hbm.at[wid], dst_v)
        plsc.subcore_barrier()

        def sidx(j):
            return src_hbm.at[pl.ds(base + j * k, k)]

        last = nchunk - 1
        if nchunk % 3 == 2 and nchunk >= 5:
            pltpu.sync_copy(sidx(0), stage_a)
            pltpu.sync_copy(sidx(1), stage_b)
            pltpu.async_copy(h_hbm.at[stage_a], rows_a, sem_a)
            pltpu.async_copy(h_hbm.at[stage_b], rows_b, sem_b)
            pltpu.async_copy(sidx(2), stage_c, sem_ic)

            def body(i, c):
                j0 = 3 * i
                pltpu.make_async_copy(sidx(j0 + 2), stage_c, sem_ic).wait()
                pltpu.make_async_copy(h_hbm.at[stage_a], rows_a, sem_a).wait()
                pltpu.async_copy(h_hbm.at[stage_c], rows_c, sem_c)
                pltpu.async_copy(sidx(j0 + 3), stage_a, sem_ia)
                pltpu.sync_copy(rows_a, acc.at[dst_v.at[j0]], add=True)
                pltpu.make_async_copy(sidx(j0 + 3), stage_a, sem_ia).wait()
                pltpu.make_async_copy(h_hbm.at[stage_b], rows_b, sem_b).wait()
                pltpu.async_copy(h_hbm.at[stage_a], rows_a, sem_a)
                pltpu.async_copy(sidx(j0 + 4), stage_b, sem_ib)
                pltpu.sync_copy(rows_b, acc.at[dst_v.at[j0 + 1]], add=True)
                pltpu.make_async_copy(sidx(j0 + 4), stage_b, sem_ib).wait()
                pltpu.make_async_copy(h_hbm.at[stage_c], rows_c, sem_c).wait()
                pltpu.async_copy(h_hbm.at[stage_b], rows_b, sem_b)
                j5 = jnp.minimum(j0 + 5, last)
                pltpu.async_copy(sidx(j5), stage_c, sem_ic)
                pltpu.sync_copy(rows_c, acc.at[dst_v.at[j0 + 2]], add=True)
                return c

            lax.fori_loop(0, (nchunk - 2) // 3, body, 0)
            pltpu.make_async_copy(h_hbm.at[stage_a], rows_a, sem_a).wait()
            pltpu.sync_copy(rows_a, acc.at[dst_v.at[last - 1]], add=True)
            pltpu.make_async_copy(h_hbm.at[stage_b], rows_b, sem_b).wait()
            pltpu.make_async_copy(sidx(last), stage_c, sem_ic).wait()
            pltpu.sync_copy(rows_b, acc.at[dst_v.at[last]], add=True)
        elif nchunk % 2 == 1 and nchunk >= 3:
            pltpu.sync_copy(sidx(0), stage_a)
            pltpu.async_copy(h_hbm.at[stage_a], rows_a, sem_a)
            pltpu.async_copy(sidx(1), stage_b, sem_ib)

            def body(i, c):
                j0 = 2 * i
                pltpu.make_async_copy(sidx(j0 + 1), stage_b, sem_ib).wait()
                pltpu.make_async_copy(h_hbm.at[stage_a], rows_a, sem_a).wait()
                pltpu.async_copy(h_hbm.at[stage_b], rows_b, sem_b)
                pltpu.async_copy(sidx(j0 + 2), stage_a, sem_ia)
                pltpu.sync_copy(rows_a, acc.at[dst_v.at[j0]], add=True)
                pltpu.make_async_copy(sidx(j0 + 2), stage_a, sem_ia).wait()
                pltpu.make_async_copy(h_hbm.at[stage_b], rows_b, sem_b).wait()
                pltpu.async_copy(h_hbm.at[stage_a], rows_a, sem_a)
                j3 = jnp.minimum(j0 + 3, last)
                pltpu.async_copy(sidx(j3), stage_b, sem_ib)
                pltpu.sync_copy(rows_b, acc.at[dst_v.at[j0 + 1]], add=True)
                return c

            lax.fori_loop(0, (nchunk - 1) // 2, body, 0)
            pltpu.make_async_copy(h_hbm.at[stage_a], rows_a, sem_a).wait()
            pltpu.make_async_copy(sidx(last), stage_b, sem_ib).wait()
            pltpu.sync_copy(rows_a, acc.at[dst_v.at[last]], add=True)
        else:
            def body(j, c):
                pltpu.sync_copy(sidx(j), stage_a)
                pltpu.async_copy(h_hbm.at[stage_a], rows_a, sem_a).wait()
                pltpu.sync_copy(rows_a, acc.at[dst_v.at[j]], add=True)
                return c

            lax.fori_loop(0, nchunk, body, 0)

        plsc.subcore_barrier()
        pltpu.sync_copy(acc.at[pl.ds(r0, rpt)], out_hbm.at[cid, pl.ds(r0, rpt)])

    return agg


def _dinv_of(deg_ref):
    dsum = jnp.sum(deg_ref[...], axis=0, keepdims=True) + 1.0
    dinv_row = lax.rsqrt(dsum)
    one = jnp.ones((1, 1), jnp.float32)
    return lax.dot_general(dinv_row, one, (((0,), (0,)), ((), ())),
                           preferred_element_type=jnp.float32)


def _mm1_body(x_ref, w_ref, deg_ref, out_ref):
    dinv = _dinv_of(deg_ref)
    out_ref[...] = jnp.dot(x_ref[...], w_ref[...],
                           preferred_element_type=jnp.float32) * dinv


def _mm2_body(a0_ref, a1_ref, hp_ref, deg_ref, b_ref, w_ref, out_ref):
    dinv = _dinv_of(deg_ref)
    h1 = (a0_ref[...] + a1_ref[...] + hp_ref[...]) * dinv + b_ref[...]
    h1 = jnp.maximum(h1, 0.0)
    out_ref[...] = jnp.dot(h1, w_ref[...],
                           preferred_element_type=jnp.float32) * dinv


def _make_z_body(dout):
    def _z_body(a0_ref, a1_ref, hp_ref, deg_ref, b_ref,
                zfull_ref, z_ref):
        dinv = _dinv_of(deg_ref)
        zf = (a0_ref[...] + a1_ref[...] + hp_ref[...]) * dinv + b_ref[...]
        zfull_ref[...] = zf
        z_ref[...] = zf[:, :dout]
    return _z_body


def _dec_body(za_ref, zb_ref, out_ref):
    s = lax.dot_general(za_ref[...], zb_ref[...],
                        (((1,), (1,)), ((), ())),
                        preferred_element_type=jnp.float32)
    out_ref[...] = 0.5 * jnp.tanh(0.5 * s) + 0.5


def kernel(x, edge_index, W1, b1, W2, b2):
    n, din = x.shape
    dhid = W1.shape[1]
    dout = W2.shape[1]
    e = edge_index.shape[1]
    ept = e // _NW
    k = _pick_chunk(ept)
    nchunk = ept // k

    n_pad = ((n + 127) // 128) * 128

    src = edge_index[0]
    dst_flat = edge_index[1]
    dst = edge_index[1].reshape(_NW, nchunk, k)
    z1r = jnp.zeros((n_pad,), jnp.float32)
    zhid = jnp.zeros((n_pad, dhid), jnp.float32)
    b1r = b1.reshape(1, dhid)
    b2r = b2.reshape(1, dout)
    d2 = max(dout, 128)
    W2p = jnp.pad(W2, ((0, 0), (0, d2 - dout))) if d2 != dout else W2
    b2p = jnp.pad(b2r, ((0, 0), (0, d2 - dout))) if d2 != dout else b2r

    deg32 = _make_deg(n_pad, ept)(dst_flat, z1r).reshape(_NW, n_pad)

    R = 1024
    grid = (pl.cdiv(n, R),)
    row = lambda i: (i, 0)
    fixed = lambda i: (0, 0)
    dspec = pl.BlockSpec((_NW, R), lambda i: (0, i))

    h1p = pl.pallas_call(
        _mm1_body,
        grid=grid,
        in_specs=[
            pl.BlockSpec((R, din), row),
            pl.BlockSpec((din, dhid), fixed),
            dspec,
        ],
        out_specs=pl.BlockSpec((R, dhid), row),
        out_shape=jax.ShapeDtypeStruct((n, dhid), jnp.float32),
    )(x, W1, deg32)

    agg1 = _make_agg(n_pad, dhid, nchunk, k)(src, dst, h1p, zhid)
    a10, a11 = agg1[0], agg1[1]

    h2p = pl.pallas_call(
        _mm2_body,
        grid=grid,
        in_specs=[
            pl.BlockSpec((R, dhid), row),
            pl.BlockSpec((R, dhid), row),
            pl.BlockSpec((R, dhid), row),
            dspec,
            pl.BlockSpec((1, dhid), fixed),
            pl.BlockSpec((dhid, d2), fixed),
        ],
        out_specs=pl.BlockSpec((R, d2), row),
        out_shape=jax.ShapeDtypeStruct((n, d2), jnp.float32),
    )(a10, a11, h1p, deg32, b1r, W2p)

    zo2 = jnp.zeros((n_pad, d2), jnp.float32)
    agg2 = _make_agg(n_pad, d2, nchunk, k)(src, dst, h2p, zo2)
    a20, a21 = agg2[0], agg2[1]

    zfull, z = pl.pallas_call(
        _make_z_body(dout),
        grid=grid,
        in_specs=[
            pl.BlockSpec((R, d2), row),
            pl.BlockSpec((R, d2), row),
            pl.BlockSpec((R, d2), row),
            dspec,
            pl.BlockSpec((1, d2), fixed),
        ],
        out_specs=[pl.BlockSpec((R, d2), row), pl.BlockSpec((R, dout), row)],
        out_shape=[jax.ShapeDtypeStruct((n, d2), jnp.float32),
                   jax.ShapeDtypeStruct((n, dout), jnp.float32)],
    )(a20, a21, h2p, deg32, b2p)

    BRI, BRJ = 1024, 2048
    adj = pl.pallas_call(
        _dec_body,
        grid=(pl.cdiv(n, BRI), pl.cdiv(n, BRJ)),
        in_specs=[
            pl.BlockSpec((BRI, d2), lambda i, j: (i, 0)),
            pl.BlockSpec((BRJ, d2), lambda i, j: (j, 0)),
        ],
        out_specs=pl.BlockSpec((BRI, BRJ), lambda i, j: (i, j)),
        out_shape=jax.ShapeDtypeStruct((n, n), jnp.float32),
    )(zfull, zfull)

    return (adj, z)


# speedup vs baseline: 2.2234x; 1.0033x over previous
import functools

import jax
import jax.numpy as jnp
from jax import lax
from jax.experimental import pallas as pl
from jax.experimental.pallas import tpu as pltpu
from jax.experimental.pallas import tpu_sc as plsc

_NC = 2
_NS = 16
_NW = _NC * _NS


def _pick_chunk(ept):
    for k in range(128, 7, -8):
        if ept % k == 0:
            return k
    raise ValueError(f"no valid chunk size for {ept} edges per tile")


def _make_deg(n_pad, ept):
    mesh = plsc.VectorSubcoreMesh(core_axis_name="c", subcore_axis_name="s")
    ngrp = ept // 16

    @functools.partial(
        pl.kernel,
        mesh=mesh,
        out_type=jax.ShapeDtypeStruct((_NW, 1, n_pad), jnp.float32),
        scratch_types=[
            pltpu.VMEM((ept,), jnp.int32),
            pltpu.VMEM((n_pad,), jnp.float32),
        ],
        compiler_params=pltpu.CompilerParams(needs_layout_passes=False),
    )
    def deg(dst_hbm, zero_hbm, out_hbm, dst_v, hist):
        cid = lax.axis_index("c")
        sid = lax.axis_index("s")
        wid = sid * _NC + cid
        pltpu.sync_copy(zero_hbm, hist)
        pltpu.sync_copy(dst_hbm.at[pl.ds(wid * ept, ept)], dst_v)
        ones16 = jnp.ones((16,), jnp.float32)

        def body(g, c):
            idx = dst_v[pl.ds(g * 16, 16)]
            plsc.addupdate_scatter(hist, [idx], ones16)
            return c

        lax.fori_loop(0, ngrp, body, 0)
        pltpu.sync_copy(hist, out_hbm.at[wid, 0])

    return deg


def _make_agg(n_pad, d, nchunk, k):
    rpt = n_pad // _NS
    mesh = plsc.VectorSubcoreMesh(core_axis_name="c", subcore_axis_name="s")

    ept = nchunk * k

    @functools.partial(
        pl.kernel,
        mesh=mesh,
        out_type=jax.ShapeDtypeStruct((_NC, n_pad, d), jnp.float32),
        scratch_types=[
            pltpu.VMEM((nchunk, k), jnp.int32),
            pltpu.VMEM((k,), jnp.int32),
            pltpu.VMEM((k,), jnp.int32),
            pltpu.VMEM((k,), jnp.int32),
            pltpu.VMEM((k, d), jnp.float32),
            pltpu.VMEM((k, d), jnp.float32),
            pltpu.VMEM((k, d), jnp.float32),
            pltpu.VMEM_SHARED((n_pad, d), jnp.float32),
            pltpu.SemaphoreType.DMA,
            pltpu.SemaphoreType.DMA,
            pltpu.SemaphoreType.DMA,
            pltpu.SemaphoreType.DMA,
            pltpu.SemaphoreType.DMA,
            pltpu.SemaphoreType.DMA,
        ],
    )
    def agg(src_hbm, dst_hbm, h_hbm, zero_hbm, out_hbm,
            dst_v, stage_a, stage_b, stage_c, rows_a, rows_b, rows_c, acc,
            sem_a, sem_b, sem_c, sem_ia, sem_ib, sem_ic):
        cid = lax.axis_index("c")
        sid = lax.axis_index("s")
        wid = sid * _NC + cid
        r0 = sid * rpt
        base = wid * ept
        pltpu.sync_copy(zero_hbm.at[pl.ds(r0, rpt)], acc.at[pl.ds(r0, rpt)])
        pltpu.sync_copy(dst_hbm.at[wid], dst_v)
        plsc.subcore_barrier()

        def sidx(j):
            return src_hbm.at[pl.ds(base + j * k, k)]

        last = nchunk - 1
        if nchunk % 3 == 2 and nchunk >= 5:
            pltpu.sync_copy(sidx(0), stage_a)
            pltpu.sync_copy(sidx(1), stage_b)
            pltpu.async_copy(h_hbm.at[stage_a], rows_a, sem_a)
            pltpu.async_copy(h_hbm.at[stage_b], rows_b, sem_b)
            pltpu.async_copy(sidx(2), stage_c, sem_ic)

            def body(i, c):
                j0 = 3 * i
                pltpu.make_async_copy(sidx(j0 + 2), stage_c, sem_ic).wait()
                pltpu.make_async_copy(h_hbm.at[stage_a], rows_a, sem_a).wait()
                pltpu.async_copy(h_hbm.at[stage_c], rows_c, sem_c)
                pltpu.async_copy(sidx(j0 + 3), stage_a, sem_ia)
                pltpu.sync_copy(rows_a, acc.at[dst_v.at[j0]], add=True)
                pltpu.make_async_copy(sidx(j0 + 3), stage_a, sem_ia).wait()
                pltpu.make_async_copy(h_hbm.at[stage_b], rows_b, sem_b).wait()
                pltpu.async_copy(h_hbm.at[stage_a], rows_a, sem_a)
                pltpu.async_copy(sidx(j0 + 4), stage_b, sem_ib)
                pltpu.sync_copy(rows_b, acc.at[dst_v.at[j0 + 1]], add=True)
                pltpu.make_async_copy(sidx(j0 + 4), stage_b, sem_ib).wait()
                pltpu.make_async_copy(h_hbm.at[stage_c], rows_c, sem_c).wait()
                pltpu.async_copy(h_hbm.at[stage_b], rows_b, sem_b)
                j5 = jnp.minimum(j0 + 5, last)
                pltpu.async_copy(sidx(j5), stage_c, sem_ic)
                pltpu.sync_copy(rows_c, acc.at[dst_v.at[j0 + 2]], add=True)
                return c

            lax.fori_loop(0, (nchunk - 2) // 3, body, 0)
            pltpu.make_async_copy(h_hbm.at[stage_a], rows_a, sem_a).wait()
            pltpu.sync_copy(rows_a, acc.at[dst_v.at[last - 1]], add=True)
            pltpu.make_async_copy(h_hbm.at[stage_b], rows_b, sem_b).wait()
            pltpu.make_async_copy(sidx(last), stage_c, sem_ic).wait()
            pltpu.sync_copy(rows_b, acc.at[dst_v.at[last]], add=True)
        elif nchunk % 2 == 1 and nchunk >= 3:
            pltpu.sync_copy(sidx(0), stage_a)
            pltpu.async_copy(h_hbm.at[stage_a], rows_a, sem_a)
            pltpu.async_copy(sidx(1), stage_b, sem_ib)

            def body(i, c):
                j0 = 2 * i
                pltpu.make_async_copy(sidx(j0 + 1), stage_b, sem_ib).wait()
                pltpu.make_async_copy(h_hbm.at[stage_a], rows_a, sem_a).wait()
                pltpu.async_copy(h_hbm.at[stage_b], rows_b, sem_b)
                pltpu.async_copy(sidx(j0 + 2), stage_a, sem_ia)
                pltpu.sync_copy(rows_a, acc.at[dst_v.at[j0]], add=True)
                pltpu.make_async_copy(sidx(j0 + 2), stage_a, sem_ia).wait()
                pltpu.make_async_copy(h_hbm.at[stage_b], rows_b, sem_b).wait()
                pltpu.async_copy(h_hbm.at[stage_a], rows_a, sem_a)
                j3 = jnp.minimum(j0 + 3, last)
                pltpu.async_copy(sidx(j3), stage_b, sem_ib)
                pltpu.sync_copy(rows_b, acc.at[dst_v.at[j0 + 1]], add=True)
                return c

            lax.fori_loop(0, (nchunk - 1) // 2, body, 0)
            pltpu.make_async_copy(h_hbm.at[stage_a], rows_a, sem_a).wait()
            pltpu.make_async_copy(sidx(last), stage_b, sem_ib).wait()
            pltpu.sync_copy(rows_a, acc.at[dst_v.at[last]], add=True)
        else:
            def body(j, c):
                pltpu.sync_copy(sidx(j), stage_a)
                pltpu.async_copy(h_hbm.at[stage_a], rows_a, sem_a).wait()
                pltpu.sync_copy(rows_a, acc.at[dst_v.at[j]], add=True)
                return c

            lax.fori_loop(0, nchunk, body, 0)

        plsc.subcore_barrier()
        pltpu.sync_copy(acc.at[pl.ds(r0, rpt)], out_hbm.at[cid, pl.ds(r0, rpt)])

    return agg


def _dinv_of(deg_ref):
    dsum = jnp.sum(deg_ref[...], axis=0, keepdims=True) + 1.0
    dinv_row = lax.rsqrt(dsum)
    one = jnp.ones((1, 1), jnp.float32)
    return lax.dot_general(dinv_row, one, (((0,), (0,)), ((), ())),
                           preferred_element_type=jnp.float32)


def _mm1_body(x_ref, w_ref, deg_ref, out_ref):
    dinv = _dinv_of(deg_ref)
    out_ref[...] = jnp.dot(x_ref[...], w_ref[...],
                           preferred_element_type=jnp.float32) * dinv


def _mm2_body(a0_ref, a1_ref, hp_ref, deg_ref, b_ref, w_ref, out_ref):
    dinv = _dinv_of(deg_ref)
    h1 = (a0_ref[...] + a1_ref[...] + hp_ref[...]) * dinv + b_ref[...]
    h1 = jnp.maximum(h1, 0.0)
    out_ref[...] = jnp.dot(h1, w_ref[...],
                           preferred_element_type=jnp.float32) * dinv


def _make_z_body(dout):
    def _z_body(a0_ref, a1_ref, hp_ref, deg_ref, b_ref,
                zfull_ref, z_ref):
        dinv = _dinv_of(deg_ref)
        zf = (a0_ref[...] + a1_ref[...] + hp_ref[...]) * dinv + b_ref[...]
        zfull_ref[...] = zf
        z_ref[...] = zf[:, :dout]
    return _z_body


def _dec_body(za_ref, zb_ref, out_ref):
    s = lax.dot_general(za_ref[...], zb_ref[...],
                        (((1,), (1,)), ((), ())),
                        preferred_element_type=jnp.float32)
    out_ref[...] = 0.5 * jnp.tanh(0.5 * s) + 0.5


def kernel(x, edge_index, W1, b1, W2, b2):
    n, din = x.shape
    dhid = W1.shape[1]
    dout = W2.shape[1]
    e = edge_index.shape[1]
    ept = e // _NW
    k = _pick_chunk(ept)
    nchunk = ept // k

    n_pad = ((n + 127) // 128) * 128

    src = edge_index[0]
    dst_flat = edge_index[1]
    dst = edge_index[1].reshape(_NW, nchunk, k)
    z1r = jnp.zeros((n_pad,), jnp.float32)
    zhid = jnp.zeros((n_pad, dhid), jnp.float32)
    b1r = b1.reshape(1, dhid)
    b2r = b2.reshape(1, dout)
    d2 = max(dout, 128)
    W2p = jnp.pad(W2, ((0, 0), (0, d2 - dout))) if d2 != dout else W2
    b2p = jnp.pad(b2r, ((0, 0), (0, d2 - dout))) if d2 != dout else b2r

    deg32 = _make_deg(n_pad, ept)(dst_flat, z1r).reshape(_NW, n_pad)

    R = 1024
    grid = (pl.cdiv(n, R),)
    row = lambda i: (i, 0)
    fixed = lambda i: (0, 0)
    dspec = pl.BlockSpec((_NW, R), lambda i: (0, i))

    h1p = pl.pallas_call(
        _mm1_body,
        grid=grid,
        in_specs=[
            pl.BlockSpec((R, din), row),
            pl.BlockSpec((din, dhid), fixed),
            dspec,
        ],
        out_specs=pl.BlockSpec((R, dhid), row),
        out_shape=jax.ShapeDtypeStruct((n, dhid), jnp.float32),
    )(x, W1, deg32)

    agg1 = _make_agg(n_pad, dhid, nchunk, k)(src, dst, h1p, zhid)
    a10, a11 = agg1[0], agg1[1]

    h2p = pl.pallas_call(
        _mm2_body,
        grid=grid,
        in_specs=[
            pl.BlockSpec((R, dhid), row),
            pl.BlockSpec((R, dhid), row),
            pl.BlockSpec((R, dhid), row),
            dspec,
            pl.BlockSpec((1, dhid), fixed),
            pl.BlockSpec((dhid, d2), fixed),
        ],
        out_specs=pl.BlockSpec((R, d2), row),
        out_shape=jax.ShapeDtypeStruct((n, d2), jnp.float32),
    )(a10, a11, h1p, deg32, b1r, W2p)

    zo2 = jnp.zeros((n_pad, d2), jnp.float32)
    agg2 = _make_agg(n_pad, d2, nchunk, k)(src, dst, h2p, zo2)
    a20, a21 = agg2[0], agg2[1]

    zfull, z = pl.pallas_call(
        _make_z_body(dout),
        grid=grid,
        in_specs=[
            pl.BlockSpec((R, d2), row),
            pl.BlockSpec((R, d2), row),
            pl.BlockSpec((R, d2), row),
            dspec,
            pl.BlockSpec((1, d2), fixed),
        ],
        out_specs=[pl.BlockSpec((R, d2), row), pl.BlockSpec((R, dout), row)],
        out_shape=[jax.ShapeDtypeStruct((n, d2), jnp.float32),
                   jax.ShapeDtypeStruct((n, dout), jnp.float32)],
    )(a20, a21, h2p, deg32, b2p)

    BRI, BRJ = 1024, 4096
    adj = pl.pallas_call(
        _dec_body,
        grid=(pl.cdiv(n, BRI), pl.cdiv(n, BRJ)),
        in_specs=[
            pl.BlockSpec((BRI, d2), lambda i, j: (i, 0)),
            pl.BlockSpec((BRJ, d2), lambda i, j: (j, 0)),
        ],
        out_specs=pl.BlockSpec((BRI, BRJ), lambda i, j: (i, j)),
        out_shape=jax.ShapeDtypeStruct((n, n), jnp.float32),
    )(zfull, zfull)

    return (adj, z)
